# baseline (speedup 1.0000x reference)
import functools

import jax
import jax.numpy as jnp
from jax import lax
from jax.experimental import pallas as pl
from jax.experimental.pallas import tpu as pltpu
from jax.experimental.pallas import tpu_sc as plsc

K_ = 10
N_ = 2048
C_ = 256
G_ = 4
RB_ = 256
QB_ = 512
F32 = jnp.float32


def _pallas_call(*args, **kwargs):
    return pl.pallas_call(*args, **kwargs)


def _dotg(a, b):
    return lax.dot_general(a, b, (((1,), (1,)), ((), ())),
                           preferred_element_type=F32)


def _leaky(x):
    return jnp.where(x >= 0, x, 0.2 * x)


def _knn_body(pts_ref, ptst_ref, idx_ref):
    g = pl.program_id(0)
    X = pts_ref[0]
    XrT = ptst_ref[0]
    sq = X[0:1, :] * X[0:1, :] + X[1:2, :] * X[1:2, :] + X[2:3, :] * X[2:3, :]
    sqr = (XrT[:, 0:1] * XrT[:, 0:1] + XrT[:, 1:2] * XrT[:, 1:2]
           + XrT[:, 2:3] * XrT[:, 2:3])
    dist = sqr + sq - 2.0 * jnp.dot(XrT, X, preferred_element_type=F32)
    iota = lax.broadcasted_iota(jnp.int32, (RB_, N_), 1)
    colidx = lax.broadcasted_iota(jnp.int32, (RB_, 16), 1)
    acc0 = jnp.zeros((RB_, 16), jnp.int32)

    def it(t, carry):
        d, acc = carry
        m = jnp.min(d, axis=1, keepdims=True)
        amin = jnp.min(jnp.where(d == m, iota, N_), axis=1, keepdims=True)
        acc = jnp.where(colidx == t, amin + g * N_, acc)
        d = jnp.where(iota == amin, jnp.inf, d)
        return d, acc

    _, acc = lax.fori_loop(0, K_ + 1, it, (dist, acc0))
    idx_ref[0] = acc


def _knn(pts, pts_t):
    return _pallas_call(
        _knn_body,
        grid=(G_, N_ // RB_),
        in_specs=[
            pl.BlockSpec((1, 8, N_), lambda g, r: (g, 0, 0)),
            pl.BlockSpec((1, RB_, 8), lambda g, r: (g, r, 0)),
        ],
        out_specs=pl.BlockSpec((1, RB_, 16), lambda g, r: (g, r, 0)),
        out_shape=jax.ShapeDtypeStruct((G_, N_, 16), jnp.int32),
    )(pts, pts_t)


def _make_sc_nbr(kp):
    info = plsc.get_sparse_core_info()
    nc, ns = info.num_cores, info.num_subcores
    nw = nc * ns
    gn = G_ * N_
    npw = gn // nw
    ch = 128
    nch = npw // ch
    mesh = plsc.VectorSubcoreMesh(core_axis_name="c", subcore_axis_name="s")

    @functools.partial(
        pl.kernel, mesh=mesh,
        out_type=jax.ShapeDtypeStruct((kp, gn, C_), F32),
        scratch_types=[
            pltpu.VMEM((-(-kp // 4) * 4, npw), jnp.int32),
            pltpu.VMEM((ch, C_), F32),
            pltpu.VMEM((ch, C_), F32),
            pltpu.SemaphoreType.DMA,
            pltpu.SemaphoreType.DMA,
            pltpu.SemaphoreType.DMA,
        ],
    )
    def sc_nbr(table_h, idx_h, out_h, idx_v, rows0, rows1, gsem, wsem0,
               wsem1):
        rows = (rows0, rows1)
        wsem = (wsem0, wsem1)
        wid = lax.axis_index("s") * nc + lax.axis_index("c")
        base = wid * npw
        pltpu.sync_copy(idx_h.at[:, pl.ds(base, npw)], idx_v.at[pl.ds(0, kp)])

        T = kp * nch

        def gstart(t):
            j, ci = divmod(t, nch)
            s = t % 2
            return pltpu.async_copy(
                table_h.at[idx_v.at[j, pl.ds(ci * ch, ch)]], rows[s], gsem)

        def wstart(t):
            j, ci = divmod(t, nch)
            s = t % 2
            return pltpu.async_copy(
                rows[s], out_h.at[j, pl.ds(base + ci * ch, ch)], wsem[s])

        w = {}
        g = gstart(0)
        for t in range(T):
            g.wait()
            w[t] = wstart(t)
            if t + 1 < T:
                if t >= 1:
                    w[t - 1].wait()
                g = gstart(t + 1)
        for t in range(max(0, T - 2), T):
            w[t].wait()

    return sc_nbr


_sc_nbr_cache = {}


def _sc_nbr(table, idx_j):
    kp = idx_j.shape[0]
    if kp not in _sc_nbr_cache:
        _sc_nbr_cache[kp] = _make_sc_nbr(kp)
    return _sc_nbr_cache[kp](table, idx_j)


NB_ = 256


def _edge_mm_body(nbr_ref, f_ref, w_ref, m_ref, sp_ref, qp_ref):
    f = f_ref[0]
    w = w_ref[...]
    m = s = q = None
    for j in range(nbr_ref.shape[0]):
        z = jnp.concatenate([f, nbr_ref[j, 0] - f], axis=1)
        e = _dotg(z, w)
        m = e if m is None else jnp.maximum(m, e)
        s = e if s is None else s + e
        q = e * e if q is None else q + e * e
    m_ref[0] = m
    sp_ref[0, 0] = jnp.sum(s, axis=0, keepdims=True)
    qp_ref[0, 0] = jnp.sum(q, axis=0, keepdims=True)


def _edge_mm(nbr, f_rows, w):
    co = w.shape[0]
    r = N_ // NB_
    kp = nbr.shape[0]
    return _pallas_call(
        _edge_mm_body,
        grid=(G_, r),
        in_specs=[
            pl.BlockSpec((kp, 1, NB_, C_), lambda g, rr: (0, g, rr, 0)),
            pl.BlockSpec((1, NB_, C_), lambda g, rr: (g, rr, 0)),
            pl.BlockSpec(w.shape, lambda g, rr: (0, 0)),
        ],
        out_specs=[
            pl.BlockSpec((1, NB_, co), lambda g, rr: (g, rr, 0)),
            pl.BlockSpec((1, 1, 1, co), lambda g, rr: (g, rr, 0, 0)),
            pl.BlockSpec((1, 1, 1, co), lambda g, rr: (g, rr, 0, 0)),
        ],
        out_shape=[
            jax.ShapeDtypeStruct((G_, N_, co), F32),
            jax.ShapeDtypeStruct((G_, r, 1, co), F32),
            jax.ShapeDtypeStruct((G_, r, 1, co), F32),
        ],
    )(nbr.reshape(kp, G_, N_, C_), f_rows, w)


def _edge_norm(ma, mb, spa_ref, spb_ref, qpa_ref, qpb_ref):
    nk = float(N_ * K_)
    mean = (jnp.sum(spa_ref[:, 0, :], axis=0, keepdims=True)
            + jnp.sum(spb_ref[:, 0, :], axis=0, keepdims=True)) / nk
    var = (jnp.sum(qpa_ref[:, 0, :], axis=0, keepdims=True)
           + jnp.sum(qpb_ref[:, 0, :], axis=0, keepdims=True)) / nk
    var = var - mean * mean
    inv = lax.rsqrt(var + 1e-5)
    return _leaky((jnp.maximum(ma, mb) - mean) * inv)


def _norm1_body(ma_ref, mb_ref, spa_ref, spb_ref, qpa_ref, qpb_ref, x1_ref):
    x1_ref[0] = _edge_norm(ma_ref[0], mb_ref[0], spa_ref[0], spb_ref[0],
                           qpa_ref[0], qpb_ref[0])


def _norm1(ha, hb):
    (ma, spa, qpa), (mb, spb, qpb) = ha, hb
    co = ma.shape[2]
    r = N_ // NB_
    xs = pl.BlockSpec((1, N_, co), lambda g: (g, 0, 0))
    ps = pl.BlockSpec((1, r, 1, co), lambda g: (g, 0, 0, 0))
    return _pallas_call(
        _norm1_body,
        grid=(G_,),
        in_specs=[xs, xs, ps, ps, ps, ps],
        out_specs=xs,
        out_shape=jax.ShapeDtypeStruct((G_, N_, co), F32),
    )(ma, mb, spa, spb, qpa, qpb)


def _mm3_body(ma_ref, mb_ref, spa_ref, spb_ref, qpa_ref, qpb_ref, x0_ref,
              x1_ref, w_ref, o_ref):
    x2 = _edge_norm(ma_ref[0], mb_ref[0], spa_ref[0], spb_ref[0],
                    qpa_ref[0], qpb_ref[0])
    z = jnp.concatenate([x0_ref[0], x1_ref[0], x2], axis=1)
    y = _dotg(z, w_ref[...])
    mean = jnp.mean(y, axis=0, keepdims=True)
    var = jnp.mean(y * y, axis=0, keepdims=True) - mean * mean
    o_ref[0] = _leaky((y - mean) * lax.rsqrt(var + 1e-5))


def _mm3(ha, hb, x0, x1, w):
    (ma, spa, qpa), (mb, spb, qpb) = ha, hb
    c2 = ma.shape[2]
    r = N_ // NB_
    c2s = pl.BlockSpec((1, N_, c2), lambda g: (g, 0, 0))
    ps = pl.BlockSpec((1, r, 1, c2), lambda g: (g, 0, 0, 0))
    cs = pl.BlockSpec((1, N_, C_), lambda g: (g, 0, 0))
    return _pallas_call(
        _mm3_body,
        grid=(G_,),
        in_specs=[c2s, c2s, ps, ps, ps, ps, cs, cs,
                  pl.BlockSpec(w.shape, lambda g: (0, 0))],
        out_specs=cs,
        out_shape=jax.ShapeDtypeStruct((G_, N_, C_), F32),
    )(ma, mb, spa, spb, qpa, qpb, x0, x1, w)


def _qkv_body(x_ref, s_ref, wq_ref, bq_ref, wk_ref, bk_ref, wv_ref, bv_ref,
              q_ref, k_ref, v_ref):
    x = x_ref[0]
    s = s_ref[0]
    q_ref[0] = _dotg(x, wq_ref[...]) + bq_ref[...]
    k_ref[0] = _dotg(s, wk_ref[...]) + bk_ref[...]
    v_ref[0] = _dotg(s, wv_ref[...]) + bv_ref[...]


def _qkv(x_rows, s_rows, wq, bq, wk, bk, wv, bv):
    g2 = x_rows.shape[0]
    wspec = pl.BlockSpec((C_, C_), lambda g: (0, 0))
    bspec = pl.BlockSpec((1, C_), lambda g: (0, 0))
    xspec = pl.BlockSpec((1, N_, C_), lambda g: (g, 0, 0))
    return _pallas_call(
        _qkv_body,
        grid=(g2,),
        in_specs=[xspec, xspec, wspec, bspec, wspec, bspec, wspec, bspec],
        out_specs=[xspec, xspec, xspec],
        out_shape=[jax.ShapeDtypeStruct((g2, N_, C_), F32)] * 3,
    )(x_rows, s_rows, wq, bq, wk, bk, wv, bv)


def _attn_body(q_ref, k_ref, v_ref, o_ref):
    q = q_ref[0]
    k = k_ref[0]
    v = v_ref[0]
    outs = []
    for h in range(4):
        qh = q[:, h * 64:(h + 1) * 64]
        kh = k[:, h * 64:(h + 1) * 64]
        s = _dotg(qh, kh) * 0.125
        mx = jnp.max(s, axis=1, keepdims=True)
        e = jnp.exp(s - mx)
        prob = e / jnp.sum(e, axis=1, keepdims=True)
        outs.append(jnp.dot(prob, v[:, h * 64:(h + 1) * 64],
                            preferred_element_type=F32))
    o_ref[0] = jnp.concatenate(outs, axis=1)


def _attn(q_rows, k_rows, v_rows):
    g2 = q_rows.shape[0]
    return _pallas_call(
        _attn_body,
        grid=(g2, N_ // QB_),
        in_specs=[
            pl.BlockSpec((1, QB_, C_), lambda g, r: (g, r, 0)),
            pl.BlockSpec((1, N_, C_), lambda g, r: (g, 0, 0)),
            pl.BlockSpec((1, N_, C_), lambda g, r: (g, 0, 0)),
        ],
        out_specs=pl.BlockSpec((1, QB_, C_), lambda g, r: (g, r, 0)),
        out_shape=jax.ShapeDtypeStruct((g2, N_, C_), F32),
    )(q_rows, k_rows, v_rows)


def _mlp_body(msg_ref, x_ref, wm_ref, bm_ref, w1_ref, b1_ref, w2_ref,
              b2_ref, o_ref):
    x = x_ref[0]
    m2 = _dotg(msg_ref[0], wm_ref[...]) + bm_ref[...]
    h = _dotg(jnp.concatenate([x, m2], axis=1), w1_ref[...]) + b1_ref[...]
    mean = jnp.mean(h, axis=0, keepdims=True)
    var = jnp.mean(h * h, axis=0, keepdims=True) - mean * mean
    h = jnp.maximum((h - mean) * lax.rsqrt(var + 1e-5), 0.0)
    o_ref[0] = x + _dotg(h, w2_ref[...]) + b2_ref[...]


def _mlp(msg_rows, x_rows, wm, bm, w1, b1, w2, b2):
    g2 = msg_rows.shape[0]
    xspec = pl.BlockSpec((1, N_, C_), lambda g: (g, 0, 0))
    return _pallas_call(
        _mlp_body,
        grid=(g2,),
        in_specs=[
            xspec, xspec,
            pl.BlockSpec(wm.shape, lambda g: (0, 0)),
            pl.BlockSpec((1, C_), lambda g: (0, 0)),
            pl.BlockSpec(w1.shape, lambda g: (0, 0)),
            pl.BlockSpec((1, 2 * C_), lambda g: (0, 0)),
            pl.BlockSpec(w2.shape, lambda g: (0, 0)),
            pl.BlockSpec((1, C_), lambda g: (0, 0)),
        ],
        out_specs=xspec,
        out_shape=jax.ShapeDtypeStruct((g2, N_, C_), F32),
    )(msg_rows, x_rows, wm, bm, w1, b1, w2, b2)


def _edge_stage(table_rows, idx_j, w):
    kh = K_ // 2
    tbl = table_rows.reshape(G_ * N_, C_)
    nbr_a = _sc_nbr(tbl, idx_j[:kh])
    nbr_b = _sc_nbr(tbl, idx_j[kh:])
    ha = _edge_mm(nbr_a, table_rows, w)
    hb = _edge_mm(nbr_b, table_rows, w)
    return ha, hb


def _self_attn(f_rows, idx_j, w1, w2, w3):
    ha, hb = _edge_stage(f_rows, idx_j, w1)
    x1 = _norm1(ha, hb)
    ha2, hb2 = _edge_stage(x1, idx_j, w2)
    return _mm3(ha2, hb2, f_rows, x1, w3)


def _att_prop(x_rows, src_rows, pw):
    q, k, v = _qkv(x_rows, src_rows, pw['wq_p'], pw['bq_p'], pw['wk_p'],
                   pw['bk_p'], pw['wv_p'], pw['bv_p'])
    msg = _attn(q, k, v)
    return _mlp(msg, x_rows, pw['wm_p'], pw['bm'], pw['mw1'],
                pw['mb1'], pw['mw2'], pw['mb2'])


def _head_perm_rows(w):
    return w.reshape(64, 4, C_).transpose(1, 0, 2).reshape(C_, C_)


def kernel(coords0, coords1, desc0, desc1, sa0_w1, sa0_w2, sa0_w3, wq, bq,
           wk, bk, wv, bv, wm, bm, mw1, mb1, mw2, mb2, sa1_w1, sa1_w2,
           sa1_w3):
    coords = jnp.concatenate([coords0, coords1], axis=0)
    pts = jnp.pad(coords, ((0, 0), (0, 5), (0, 0)))
    pts_t = pts.transpose(0, 2, 1)
    idx = _knn(pts, pts_t)
    idx_j = idx[:, :, 1:K_ + 1].reshape(G_ * N_, K_).transpose(1, 0)

    f_rows = jnp.concatenate([desc0, desc1], axis=0).transpose(0, 2, 1)

    pw = {
        'wq_p': _head_perm_rows(wq),
        'wk_p': _head_perm_rows(wk),
        'wv_p': _head_perm_rows(wv),
        'bq_p': bq.reshape(64, 4).T.reshape(1, C_),
        'bk_p': bk.reshape(64, 4).T.reshape(1, C_),
        'bv_p': bv.reshape(64, 4).T.reshape(1, C_),
        'wm_p': wm.reshape(C_, 64, 4).transpose(0, 2, 1).reshape(C_, C_),
        'bm': bm.reshape(1, C_),
        'mw1': mw1,
        'mb1': mb1.reshape(1, 2 * C_),
        'mw2': mw2,
        'mb2': mb2.reshape(1, C_),
    }

    d = _self_attn(f_rows, idx_j, sa0_w1, sa0_w2, sa0_w3)
    d0, d1 = d[:2], d[2:]
    d0 = _att_prop(d0, d1, pw)
    d1 = _att_prop(d1, d0, pw)
    d = jnp.concatenate([d0, d1], axis=0)
    d = _self_attn(d, idx_j, sa1_w1, sa1_w2, sa1_w3)
    return (d[:2].transpose(0, 2, 1), d[2:].transpose(0, 2, 1))


# speedup vs baseline: 1.0396x; 1.0396x over previous
import functools

import jax
import jax.numpy as jnp
from jax import lax
from jax.experimental import pallas as pl
from jax.experimental.pallas import tpu as pltpu
from jax.experimental.pallas import tpu_sc as plsc

K_ = 10
N_ = 2048
C_ = 256
G_ = 4
RB_ = 256
QB_ = 512
F32 = jnp.float32


def _pallas_call(*args, **kwargs):
    return pl.pallas_call(*args, **kwargs)


def _dotg(a, b):
    return lax.dot_general(a, b, (((1,), (1,)), ((), ())),
                           preferred_element_type=F32)


def _leaky(x):
    return jnp.where(x >= 0, x, 0.2 * x)


def _knn_body(pts_ref, ptst_ref, idx_ref):
    g = pl.program_id(0)
    X = pts_ref[0]
    XrT = ptst_ref[0]
    sq = X[0:1, :] * X[0:1, :] + X[1:2, :] * X[1:2, :] + X[2:3, :] * X[2:3, :]
    sqr = (XrT[:, 0:1] * XrT[:, 0:1] + XrT[:, 1:2] * XrT[:, 1:2]
           + XrT[:, 2:3] * XrT[:, 2:3])
    dist = sqr + sq - 2.0 * jnp.dot(XrT, X, preferred_element_type=F32)
    iota = lax.broadcasted_iota(jnp.int32, (RB_, N_), 1)
    colidx = lax.broadcasted_iota(jnp.int32, (RB_, 16), 1)
    acc0 = jnp.zeros((RB_, 16), jnp.int32)

    def it(t, carry):
        d, acc = carry
        m = jnp.min(d, axis=1, keepdims=True)
        amin = jnp.min(jnp.where(d == m, iota, N_), axis=1, keepdims=True)
        acc = jnp.where(colidx == t, amin + g * N_, acc)
        d = jnp.where(iota == amin, jnp.inf, d)
        return d, acc

    _, acc = lax.fori_loop(0, K_ + 1, it, (dist, acc0))
    idx_ref[0] = acc


def _knn(pts, pts_t):
    return _pallas_call(
        _knn_body,
        grid=(G_, N_ // RB_),
        in_specs=[
            pl.BlockSpec((1, 8, N_), lambda g, r: (g, 0, 0)),
            pl.BlockSpec((1, RB_, 8), lambda g, r: (g, r, 0)),
        ],
        out_specs=pl.BlockSpec((1, RB_, 16), lambda g, r: (g, r, 0)),
        out_shape=jax.ShapeDtypeStruct((G_, N_, 16), jnp.int32),
    )(pts, pts_t)


def _make_sc_nbr(kp):
    info = plsc.get_sparse_core_info()
    nc, ns = info.num_cores, info.num_subcores
    nw = nc * ns
    gn = G_ * N_
    npw = gn // nw
    ch = 128
    nch = npw // ch
    mesh = plsc.VectorSubcoreMesh(core_axis_name="c", subcore_axis_name="s")

    @functools.partial(
        pl.kernel, mesh=mesh,
        out_type=jax.ShapeDtypeStruct((kp, gn, C_), F32),
        scratch_types=[
            pltpu.VMEM((kp, npw), jnp.int32),
            pltpu.VMEM((ch, C_), F32),
            pltpu.VMEM((ch, C_), F32),
            pltpu.SemaphoreType.DMA,
            pltpu.SemaphoreType.DMA,
            pltpu.SemaphoreType.DMA,
        ],
    )
    def sc_nbr(table_h, idx_h, out_h, idx_v, rows0, rows1, gsem, wsem0,
               wsem1):
        rows = (rows0, rows1)
        wsem = (wsem0, wsem1)
        wid = lax.axis_index("s") * nc + lax.axis_index("c")
        base = wid * npw
        pltpu.sync_copy(idx_h.at[:, pl.ds(base, npw)], idx_v)

        T = kp * nch

        def gstart(t):
            j, ci = divmod(t, nch)
            s = t % 2
            return pltpu.async_copy(
                table_h.at[idx_v.at[j, pl.ds(ci * ch, ch)]], rows[s], gsem)

        def wstart(t):
            j, ci = divmod(t, nch)
            s = t % 2
            return pltpu.async_copy(
                rows[s], out_h.at[j, pl.ds(base + ci * ch, ch)], wsem[s])

        w = {}
        g = gstart(0)
        for t in range(T):
            g.wait()
            w[t] = wstart(t)
            if t + 1 < T:
                if t >= 1:
                    w[t - 1].wait()
                g = gstart(t + 1)
        for t in range(max(0, T - 2), T):
            w[t].wait()

    return sc_nbr


_sc_nbr_cache = {}


def _sc_nbr(table, idx_j):
    kp = idx_j.shape[0]
    if kp not in _sc_nbr_cache:
        _sc_nbr_cache[kp] = _make_sc_nbr(kp)
    return _sc_nbr_cache[kp](table, idx_j)


NB_ = 256


def _edge_mm_body(nbr_ref, f_ref, w_ref, m_ref, sp_ref, qp_ref):
    f = f_ref[0]
    w = w_ref[...]
    m = s = q = None
    for j in range(nbr_ref.shape[0]):
        z = jnp.concatenate([f, nbr_ref[j, 0] - f], axis=1)
        e = _dotg(z, w)
        m = e if m is None else jnp.maximum(m, e)
        s = e if s is None else s + e
        q = e * e if q is None else q + e * e
    m_ref[0] = m
    sp_ref[0, 0] = jnp.sum(s, axis=0, keepdims=True)
    qp_ref[0, 0] = jnp.sum(q, axis=0, keepdims=True)


def _edge_mm(nbr, f_rows, w):
    co = w.shape[0]
    r = N_ // NB_
    kp = nbr.shape[0]
    return _pallas_call(
        _edge_mm_body,
        grid=(G_, r),
        in_specs=[
            pl.BlockSpec((kp, 1, NB_, C_), lambda g, rr: (0, g, rr, 0)),
            pl.BlockSpec((1, NB_, C_), lambda g, rr: (g, rr, 0)),
            pl.BlockSpec(w.shape, lambda g, rr: (0, 0)),
        ],
        out_specs=[
            pl.BlockSpec((1, NB_, co), lambda g, rr: (g, rr, 0)),
            pl.BlockSpec((1, 1, 1, co), lambda g, rr: (g, rr, 0, 0)),
            pl.BlockSpec((1, 1, 1, co), lambda g, rr: (g, rr, 0, 0)),
        ],
        out_shape=[
            jax.ShapeDtypeStruct((G_, N_, co), F32),
            jax.ShapeDtypeStruct((G_, r, 1, co), F32),
            jax.ShapeDtypeStruct((G_, r, 1, co), F32),
        ],
    )(nbr.reshape(kp, G_, N_, C_), f_rows, w)


def _edge_norm(m, sp_ref, qp_ref):
    nk = float(N_ * K_)
    mean = jnp.sum(sp_ref[:, 0, :], axis=0, keepdims=True) / nk
    var = jnp.sum(qp_ref[:, 0, :], axis=0, keepdims=True) / nk - mean * mean
    inv = lax.rsqrt(var + 1e-5)
    return _leaky((m - mean) * inv)


def _norm1_body(m_ref, sp_ref, qp_ref, x1_ref):
    x1_ref[0] = _edge_norm(m_ref[0], sp_ref[0], qp_ref[0])


def _norm1(h):
    m1, sp1, qp1 = h
    co = m1.shape[2]
    r = N_ // NB_
    xs = pl.BlockSpec((1, N_, co), lambda g: (g, 0, 0))
    ps = pl.BlockSpec((1, r, 1, co), lambda g: (g, 0, 0, 0))
    return _pallas_call(
        _norm1_body,
        grid=(G_,),
        in_specs=[xs, ps, ps],
        out_specs=xs,
        out_shape=jax.ShapeDtypeStruct((G_, N_, co), F32),
    )(m1, sp1, qp1)


def _mm3_body(m2_ref, sp2_ref, qp2_ref, x0_ref, x1_ref, w_ref, o_ref):
    x2 = _edge_norm(m2_ref[0], sp2_ref[0], qp2_ref[0])
    z = jnp.concatenate([x0_ref[0], x1_ref[0], x2], axis=1)
    y = _dotg(z, w_ref[...])
    mean = jnp.mean(y, axis=0, keepdims=True)
    var = jnp.mean(y * y, axis=0, keepdims=True) - mean * mean
    o_ref[0] = _leaky((y - mean) * lax.rsqrt(var + 1e-5))


def _mm3(h, x0, x1, w):
    m2, sp2, qp2 = h
    c2 = m2.shape[2]
    r = N_ // NB_
    c2s = pl.BlockSpec((1, N_, c2), lambda g: (g, 0, 0))
    ps = pl.BlockSpec((1, r, 1, c2), lambda g: (g, 0, 0, 0))
    cs = pl.BlockSpec((1, N_, C_), lambda g: (g, 0, 0))
    return _pallas_call(
        _mm3_body,
        grid=(G_,),
        in_specs=[c2s, ps, ps, cs, cs,
                  pl.BlockSpec(w.shape, lambda g: (0, 0))],
        out_specs=cs,
        out_shape=jax.ShapeDtypeStruct((G_, N_, C_), F32),
    )(m2, sp2, qp2, x0, x1, w)


def _qkv_body(x_ref, s_ref, wq_ref, bq_ref, wk_ref, bk_ref, wv_ref, bv_ref,
              q_ref, k_ref, v_ref):
    x = x_ref[0]
    s = s_ref[0]
    q_ref[0] = _dotg(x, wq_ref[...]) + bq_ref[...]
    k_ref[0] = _dotg(s, wk_ref[...]) + bk_ref[...]
    v_ref[0] = _dotg(s, wv_ref[...]) + bv_ref[...]


def _qkv(x_rows, s_rows, wq, bq, wk, bk, wv, bv):
    g2 = x_rows.shape[0]
    wspec = pl.BlockSpec((C_, C_), lambda g: (0, 0))
    bspec = pl.BlockSpec((1, C_), lambda g: (0, 0))
    xspec = pl.BlockSpec((1, N_, C_), lambda g: (g, 0, 0))
    return _pallas_call(
        _qkv_body,
        grid=(g2,),
        in_specs=[xspec, xspec, wspec, bspec, wspec, bspec, wspec, bspec],
        out_specs=[xspec, xspec, xspec],
        out_shape=[jax.ShapeDtypeStruct((g2, N_, C_), F32)] * 3,
    )(x_rows, s_rows, wq, bq, wk, bk, wv, bv)


def _attn_body(q_ref, k_ref, v_ref, o_ref):
    q = q_ref[0]
    k = k_ref[0]
    v = v_ref[0]
    outs = []
    for h in range(4):
        qh = q[:, h * 64:(h + 1) * 64]
        kh = k[:, h * 64:(h + 1) * 64]
        s = _dotg(qh, kh) * 0.125
        mx = jnp.max(s, axis=1, keepdims=True)
        e = jnp.exp(s - mx)
        prob = e / jnp.sum(e, axis=1, keepdims=True)
        outs.append(jnp.dot(prob, v[:, h * 64:(h + 1) * 64],
                            preferred_element_type=F32))
    o_ref[0] = jnp.concatenate(outs, axis=1)


def _attn(q_rows, k_rows, v_rows):
    g2 = q_rows.shape[0]
    return _pallas_call(
        _attn_body,
        grid=(g2, N_ // QB_),
        in_specs=[
            pl.BlockSpec((1, QB_, C_), lambda g, r: (g, r, 0)),
            pl.BlockSpec((1, N_, C_), lambda g, r: (g, 0, 0)),
            pl.BlockSpec((1, N_, C_), lambda g, r: (g, 0, 0)),
        ],
        out_specs=pl.BlockSpec((1, QB_, C_), lambda g, r: (g, r, 0)),
        out_shape=jax.ShapeDtypeStruct((g2, N_, C_), F32),
    )(q_rows, k_rows, v_rows)


def _mlp_body(msg_ref, x_ref, wm_ref, bm_ref, w1_ref, b1_ref, w2_ref,
              b2_ref, o_ref):
    x = x_ref[0]
    m2 = _dotg(msg_ref[0], wm_ref[...]) + bm_ref[...]
    h = _dotg(jnp.concatenate([x, m2], axis=1), w1_ref[...]) + b1_ref[...]
    mean = jnp.mean(h, axis=0, keepdims=True)
    var = jnp.mean(h * h, axis=0, keepdims=True) - mean * mean
    h = jnp.maximum((h - mean) * lax.rsqrt(var + 1e-5), 0.0)
    o_ref[0] = x + _dotg(h, w2_ref[...]) + b2_ref[...]


def _mlp(msg_rows, x_rows, wm, bm, w1, b1, w2, b2):
    g2 = msg_rows.shape[0]
    xspec = pl.BlockSpec((1, N_, C_), lambda g: (g, 0, 0))
    return _pallas_call(
        _mlp_body,
        grid=(g2,),
        in_specs=[
            xspec, xspec,
            pl.BlockSpec(wm.shape, lambda g: (0, 0)),
            pl.BlockSpec((1, C_), lambda g: (0, 0)),
            pl.BlockSpec(w1.shape, lambda g: (0, 0)),
            pl.BlockSpec((1, 2 * C_), lambda g: (0, 0)),
            pl.BlockSpec(w2.shape, lambda g: (0, 0)),
            pl.BlockSpec((1, C_), lambda g: (0, 0)),
        ],
        out_specs=xspec,
        out_shape=jax.ShapeDtypeStruct((g2, N_, C_), F32),
    )(msg_rows, x_rows, wm, bm, w1, b1, w2, b2)


def _self_attn(f_rows, idx_j, w1, w2, w3):
    tbl = f_rows.reshape(G_ * N_, C_)
    h1 = _edge_mm(_sc_nbr(tbl, idx_j), f_rows, w1)
    x1 = _norm1(h1)
    h2 = _edge_mm(_sc_nbr(x1.reshape(G_ * N_, C_), idx_j), x1, w2)
    return _mm3(h2, f_rows, x1, w3)


def _att_prop(x_rows, src_rows, pw):
    q, k, v = _qkv(x_rows, src_rows, pw['wq_p'], pw['bq_p'], pw['wk_p'],
                   pw['bk_p'], pw['wv_p'], pw['bv_p'])
    msg = _attn(q, k, v)
    return _mlp(msg, x_rows, pw['wm_p'], pw['bm'], pw['mw1'],
                pw['mb1'], pw['mw2'], pw['mb2'])


def _head_perm_rows(w):
    return w.reshape(64, 4, C_).transpose(1, 0, 2).reshape(C_, C_)


def kernel(coords0, coords1, desc0, desc1, sa0_w1, sa0_w2, sa0_w3, wq, bq,
           wk, bk, wv, bv, wm, bm, mw1, mb1, mw2, mb2, sa1_w1, sa1_w2,
           sa1_w3):
    coords = jnp.concatenate([coords0, coords1], axis=0)
    pts = jnp.pad(coords, ((0, 0), (0, 5), (0, 0)))
    pts_t = pts.transpose(0, 2, 1)
    idx = _knn(pts, pts_t)
    idx_j = idx[:, :, 1:K_ + 1].reshape(G_ * N_, K_).transpose(1, 0)

    f_rows = jnp.concatenate([desc0, desc1], axis=0).transpose(0, 2, 1)

    pw = {
        'wq_p': _head_perm_rows(wq),
        'wk_p': _head_perm_rows(wk),
        'wv_p': _head_perm_rows(wv),
        'bq_p': bq.reshape(64, 4).T.reshape(1, C_),
        'bk_p': bk.reshape(64, 4).T.reshape(1, C_),
        'bv_p': bv.reshape(64, 4).T.reshape(1, C_),
        'wm_p': wm.reshape(C_, 64, 4).transpose(0, 2, 1).reshape(C_, C_),
        'bm': bm.reshape(1, C_),
        'mw1': mw1,
        'mb1': mb1.reshape(1, 2 * C_),
        'mw2': mw2,
        'mb2': mb2.reshape(1, C_),
    }

    d = _self_attn(f_rows, idx_j, sa0_w1, sa0_w2, sa0_w3)
    d0, d1 = d[:2], d[2:]
    d0 = _att_prop(d0, d1, pw)
    d1 = _att_prop(d1, d0, pw)
    d = jnp.concatenate([d0, d1], axis=0)
    d = _self_attn(d, idx_j, sa1_w1, sa1_w2, sa1_w3)
    return (d[:2].transpose(0, 2, 1), d[2:].transpose(0, 2, 1))
